# baseline (device time: 15100 ns/iter reference)
import jax
import jax.numpy as jnp
from jax import lax
from jax.experimental import pallas as pl
from jax.experimental.pallas import tpu as pltpu

F_ROWS = 192
CF = 6


def kernel(x):
    m_per, n = x.shape
    d_rows = m_per - 2 * F_ROWS
    d0 = 2 * F_ROWS
    rows_c = F_ROWS // CF

    def body(x_ref, out_ref, fbuf, zs_sems, zr_sems, fs_sems, fr_sems,
             lc_sems, local_sem):
        my_x = lax.axis_index("x")
        my_y = lax.axis_index("y")
        my_z = lax.axis_index("z")
        p = (my_x + my_y) % 2
        z_nbr = (my_x, my_y, 1 - my_z)
        x_nbr = (1 - my_x, my_y, my_z)
        y_nbr = (my_x, 1 - my_y, my_z)

        barrier_sem = pltpu.get_barrier_semaphore()
        for nbr in (z_nbr, x_nbr, y_nbr):
            pl.semaphore_signal(
                barrier_sem, inc=1, device_id=nbr,
                device_id_type=pl.DeviceIdType.MESH,
            )
        pl.semaphore_wait(barrier_sem, 3)

        f0 = p * F_ROWS
        send_base = my_z * m_per
        recv_base = (1 - my_z) * m_per

        z_rdmas = []
        for c in range(CF):
            rdma = pltpu.make_async_remote_copy(
                src_ref=x_ref.at[pl.ds(f0 + c * rows_c, rows_c), :],
                dst_ref=fbuf.at[pl.ds(c * rows_c, rows_c), :],
                send_sem=zs_sems.at[c],
                recv_sem=zr_sems.at[c],
                device_id=z_nbr,
                device_id_type=pl.DeviceIdType.MESH,
            )
            rdma.start()
            z_rdmas.append(rdma)
        d_rdma = pltpu.make_async_remote_copy(
            src_ref=x_ref.at[pl.ds(d0, d_rows), :],
            dst_ref=out_ref.at[pl.ds(send_base + d0, d_rows), :],
            send_sem=zs_sems.at[CF],
            recv_sem=zr_sems.at[CF],
            device_id=z_nbr,
            device_id_type=pl.DeviceIdType.MESH,
        )
        d_rdma.start()

        local = pltpu.make_async_copy(
            x_ref, out_ref.at[pl.ds(send_base, m_per), :], local_sem
        )
        local.start()

        f_rdmas = []
        lcopies = []
        for c in range(CF):
            z_rdmas[c].wait_recv()
            nbr = x_nbr if c % 2 == 0 else y_nbr
            src = fbuf.at[pl.ds(c * rows_c, rows_c), :]
            dst = out_ref.at[pl.ds(recv_base + f0 + c * rows_c, rows_c), :]
            rdma = pltpu.make_async_remote_copy(
                src_ref=src,
                dst_ref=dst,
                send_sem=fs_sems.at[c],
                recv_sem=fr_sems.at[c],
                device_id=nbr,
                device_id_type=pl.DeviceIdType.MESH,
            )
            rdma.start()
            f_rdmas.append(rdma)
            lc = pltpu.make_async_copy(src, dst, lc_sems.at[c])
            lc.start()
            lcopies.append(lc)

        for rdma in z_rdmas:
            rdma.wait_send()
        d_rdma.wait()
        for rdma in f_rdmas:
            rdma.wait()
        for lc in lcopies:
            lc.wait()
        local.wait()

    return pl.pallas_call(
        body,
        out_shape=jax.ShapeDtypeStruct((2 * m_per, n), x.dtype),
        in_specs=[pl.BlockSpec(memory_space=pltpu.VMEM)],
        out_specs=pl.BlockSpec(memory_space=pl.ANY),
        scratch_shapes=[
            pltpu.VMEM((F_ROWS, n), x.dtype),
            pltpu.SemaphoreType.DMA((CF + 1,)),
            pltpu.SemaphoreType.DMA((CF + 1,)),
            pltpu.SemaphoreType.DMA((CF,)),
            pltpu.SemaphoreType.DMA((CF,)),
            pltpu.SemaphoreType.DMA((CF,)),
            pltpu.SemaphoreType.DMA,
        ],
        compiler_params=pltpu.CompilerParams(collective_id=0),
    )(x)


# device time: 13746 ns/iter; 1.0985x vs baseline; 1.0985x over previous
import jax
import jax.numpy as jnp
from jax import lax
from jax.experimental import pallas as pl
from jax.experimental.pallas import tpu as pltpu

F_ROWS = 192
CF = 6


def kernel(x):
    m_per, n = x.shape
    d_rows = m_per - 2 * F_ROWS
    d0 = 2 * F_ROWS
    rows_c = F_ROWS // CF

    def body(x_ref, out_ref, zs_sems, zr_sems, fs_sems, fr_sems):
        my_x = lax.axis_index("x")
        my_y = lax.axis_index("y")
        my_z = lax.axis_index("z")
        p = (my_x + my_y) % 2
        z_nbr = (my_x, my_y, 1 - my_z)
        x_nbr = (1 - my_x, my_y, my_z)
        y_nbr = (my_x, 1 - my_y, my_z)

        barrier_sem = pltpu.get_barrier_semaphore()
        pl.semaphore_wait(barrier_sem, 0)

        f0 = p * F_ROWS
        send_base = my_z * m_per
        recv_base = (1 - my_z) * m_per

        z_rdmas = []
        for c in range(CF):
            off = f0 + c * rows_c
            rdma = pltpu.make_async_remote_copy(
                src_ref=x_ref.at[pl.ds(off, rows_c), :],
                dst_ref=out_ref.at[pl.ds(send_base + off, rows_c), :],
                send_sem=zs_sems.at[c],
                recv_sem=zr_sems.at[c],
                device_id=z_nbr,
                device_id_type=pl.DeviceIdType.MESH,
            )
            rdma.start()
            z_rdmas.append(rdma)
        d_rdma = pltpu.make_async_remote_copy(
            src_ref=x_ref.at[pl.ds(d0, d_rows), :],
            dst_ref=out_ref.at[pl.ds(send_base + d0, d_rows), :],
            send_sem=zs_sems.at[CF],
            recv_sem=zr_sems.at[CF],
            device_id=z_nbr,
            device_id_type=pl.DeviceIdType.MESH,
        )
        d_rdma.start()

        out_ref[pl.ds(send_base, m_per), :] = x_ref[:, :]

        f_rdmas = []
        for c in range(CF):
            z_rdmas[c].wait_recv()
            nbr = x_nbr if c % 2 == 0 else y_nbr
            region = pl.ds(recv_base + f0 + c * rows_c, rows_c)
            rdma = pltpu.make_async_remote_copy(
                src_ref=out_ref.at[region, :],
                dst_ref=out_ref.at[region, :],
                send_sem=fs_sems.at[c],
                recv_sem=fr_sems.at[c],
                device_id=nbr,
                device_id_type=pl.DeviceIdType.MESH,
            )
            rdma.start()
            f_rdmas.append(rdma)

        for rdma in z_rdmas:
            rdma.wait_send()
        d_rdma.wait()
        for rdma in f_rdmas:
            rdma.wait()

    return pl.pallas_call(
        body,
        out_shape=jax.ShapeDtypeStruct((2 * m_per, n), x.dtype),
        in_specs=[pl.BlockSpec(memory_space=pltpu.VMEM)],
        out_specs=pl.BlockSpec(memory_space=pltpu.VMEM),
        scratch_shapes=[
            pltpu.SemaphoreType.DMA((CF + 1,)),
            pltpu.SemaphoreType.DMA((CF + 1,)),
            pltpu.SemaphoreType.DMA((CF,)),
            pltpu.SemaphoreType.DMA((CF,)),
        ],
        compiler_params=pltpu.CompilerParams(collective_id=0),
    )(x)
